# baseline (speedup 1.0000x reference)
import functools

import jax
import jax.numpy as jnp
from jax import lax
from jax.experimental import pallas as pl
from jax.experimental.pallas import tpu as pltpu
from jax.experimental.pallas import tpu_sc as plsc


def _sc_gather3(emb_page, emb_mat, emb_name, page_idx, mat_idx, name_idx):
    info = plsc.get_sparse_core_info()
    nw = info.num_cores * info.num_subcores
    b = page_idx.shape[0]
    ed = emb_page.shape[1]
    bpw = b // nw
    mesh = plsc.VectorSubcoreMesh(core_axis_name="c", subcore_axis_name="s")

    @functools.partial(
        pl.kernel,
        mesh=mesh,
        out_type=(
            jax.ShapeDtypeStruct((b, ed), jnp.float32),
            jax.ShapeDtypeStruct((b, ed), jnp.float32),
            jax.ShapeDtypeStruct((b, ed), jnp.float32),
        ),
        scratch_types=[
            pltpu.VMEM((bpw,), jnp.int32),
            pltpu.VMEM((bpw,), jnp.int32),
            pltpu.VMEM((bpw,), jnp.int32),
            pltpu.VMEM((bpw, ed), jnp.float32),
            pltpu.VMEM((bpw, ed), jnp.float32),
            pltpu.VMEM((bpw, ed), jnp.float32),
            pltpu.SemaphoreType.DMA,
            pltpu.SemaphoreType.DMA,
            pltpu.SemaphoreType.DMA,
        ],
        compiler_params=pltpu.CompilerParams(use_tc_tiling_on_sc=False),
    )
    def gather_k(pg_t, mt_t, nm_t, pg_i, mt_i, nm_i, pe_o, me_o, ne_o,
                 iv0, iv1, iv2, rv0, rv1, rv2, s0, s1, s2):
        wid = lax.axis_index("s") * info.num_cores + lax.axis_index("c")
        base = wid * bpw
        pltpu.sync_copy(pg_i.at[pl.ds(base, bpw)], iv0)
        pltpu.sync_copy(mt_i.at[pl.ds(base, bpw)], iv1)
        pltpu.sync_copy(nm_i.at[pl.ds(base, bpw)], iv2)
        c0 = pltpu.async_copy(pg_t.at[iv0], rv0, s0)
        c1 = pltpu.async_copy(mt_t.at[iv1], rv1, s1)
        c2 = pltpu.async_copy(nm_t.at[iv2], rv2, s2)
        c0.wait()
        c1.wait()
        c2.wait()
        pltpu.sync_copy(rv0, pe_o.at[pl.ds(base, bpw)])
        pltpu.sync_copy(rv1, me_o.at[pl.ds(base, bpw)])
        pltpu.sync_copy(rv2, ne_o.at[pl.ds(base, bpw)])

    return gather_k(emb_page, emb_mat, emb_name, page_idx, mat_idx, name_idx)


def _tc_heads(x, pe, me, ne, pos, dirc, evt, w_small, b_small, w_name, b_name):
    b = x.shape[0]
    bm = 512
    n_small = w_small.shape[1]
    n_name = w_name.shape[1]

    def body(x_r, pe_r, me_r, ne_r, pos_r, dir_r, evt_r,
             ws_r, bs_r, wn_r, bn_r, os_r, on_r):
        xb = x_r[...]
        tb = jnp.concatenate([pos_r[...], dir_r[...], evt_r[...]], axis=1)
        peb = pe_r[...]
        meb = me_r[...]
        neb = ne_r[...]

        def heads(w_r, bias_r, n):
            acc = jnp.dot(xb, w_r[0:64, :], preferred_element_type=jnp.float32)
            acc += jnp.dot(peb, w_r[64:80, :], preferred_element_type=jnp.float32)
            acc += jnp.dot(meb, w_r[80:96, :], preferred_element_type=jnp.float32)
            acc += jnp.dot(neb, w_r[96:112, :], preferred_element_type=jnp.float32)
            acc += jnp.dot(tb, w_r[112:117, :], preferred_element_type=jnp.float32)
            return acc + bias_r[...]

        os_r[...] = heads(ws_r, bs_r, n_small)
        on_r[...] = heads(wn_r, bn_r, n_name)

    grid = (b // bm,)
    row = lambda i: (i, 0)
    rep = lambda i: (0, 0)
    out_small, out_name = pl.pallas_call(
        body,
        grid=grid,
        in_specs=[
            pl.BlockSpec((bm, 64), row),
            pl.BlockSpec((bm, 16), row),
            pl.BlockSpec((bm, 16), row),
            pl.BlockSpec((bm, 16), row),
            pl.BlockSpec((bm, 3), row),
            pl.BlockSpec((bm, 1), row),
            pl.BlockSpec((bm, 1), row),
            pl.BlockSpec((117, n_small), rep),
            pl.BlockSpec((1, n_small), rep),
            pl.BlockSpec((117, n_name), rep),
            pl.BlockSpec((1, n_name), rep),
        ],
        out_specs=[
            pl.BlockSpec((bm, n_small), row),
            pl.BlockSpec((bm, n_name), row),
        ],
        out_shape=[
            jax.ShapeDtypeStruct((b, n_small), jnp.float32),
            jax.ShapeDtypeStruct((b, n_name), jnp.float32),
        ],
        compiler_params=pltpu.CompilerParams(
            dimension_semantics=("arbitrary",),
        ),
    )(x, pe, me, ne, pos, dirc, evt, w_small, b_small, w_name, b_name)
    return out_small, out_name


def kernel(x, PageName, MaterialName, Name, Position, Direction, event_type,
           emb_page, emb_mat, emb_name,
           W_time, b_time, W_inputs, b_inputs, W_position, b_position,
           W_velocity, b_velocity, W_event_type, b_event_type,
           W_block_name, b_block_name, W_block_position, b_block_position,
           W_block_direction, b_block_direction):
    pe, me, ne = _sc_gather3(emb_page, emb_mat, emb_name,
                             PageName, MaterialName, Name)

    w_small = jnp.concatenate(
        [W_time, W_inputs, W_position, W_velocity, W_event_type,
         W_block_position, W_block_direction], axis=1)
    b_small = jnp.concatenate(
        [b_time, b_inputs, b_position, b_velocity, b_event_type,
         b_block_position, b_block_direction], axis=0)[None, :]

    out_small, out_name = _tc_heads(
        x, pe, me, ne,
        Position, Direction[:, None], event_type[:, None],
        w_small, b_small, W_block_name, b_block_name[None, :])

    out_time = out_small[:, 0:1]
    out_inputs = out_small[:, 1:4]
    out_position = out_small[:, 4:7]
    out_velocity = out_small[:, 7:10]
    out_event_type = out_small[:, 10:18]
    out_block_position = out_small[:, 18:21]
    out_block_direction = out_small[:, 21:25]
    return (out_time, out_inputs, out_position, out_velocity, out_event_type,
            out_name, out_block_position, out_block_direction)


# speedup vs baseline: 1.0398x; 1.0398x over previous
import functools

import jax
import jax.numpy as jnp
from jax import lax
from jax.experimental import pallas as pl
from jax.experimental.pallas import tpu as pltpu
from jax.experimental.pallas import tpu_sc as plsc


def _sc_gather3(emb_page, emb_mat, emb_name, page_idx, mat_idx, name_idx):
    info = plsc.get_sparse_core_info()
    nw = info.num_cores * info.num_subcores
    b = page_idx.shape[0]
    ed = emb_page.shape[1]
    bpw = b // nw
    mesh = plsc.VectorSubcoreMesh(core_axis_name="c", subcore_axis_name="s")

    @functools.partial(
        pl.kernel,
        mesh=mesh,
        out_type=(
            jax.ShapeDtypeStruct((b, ed), jnp.float32),
            jax.ShapeDtypeStruct((b, ed), jnp.float32),
            jax.ShapeDtypeStruct((b, ed), jnp.float32),
        ),
        scratch_types=[
            pltpu.VMEM((bpw,), jnp.int32),
            pltpu.VMEM((bpw,), jnp.int32),
            pltpu.VMEM((bpw,), jnp.int32),
            pltpu.VMEM((bpw, ed), jnp.float32),
            pltpu.VMEM((bpw, ed), jnp.float32),
            pltpu.VMEM((bpw, ed), jnp.float32),
            pltpu.SemaphoreType.DMA,
            pltpu.SemaphoreType.DMA,
            pltpu.SemaphoreType.DMA,
        ],
        compiler_params=pltpu.CompilerParams(use_tc_tiling_on_sc=False),
    )
    def gather_k(pg_t, mt_t, nm_t, pg_i, mt_i, nm_i, pe_o, me_o, ne_o,
                 iv0, iv1, iv2, rv0, rv1, rv2, s0, s1, s2):
        wid = lax.axis_index("s") * info.num_cores + lax.axis_index("c")
        base = wid * bpw
        pltpu.sync_copy(pg_i.at[pl.ds(base, bpw)], iv0)
        pltpu.sync_copy(mt_i.at[pl.ds(base, bpw)], iv1)
        pltpu.sync_copy(nm_i.at[pl.ds(base, bpw)], iv2)
        c0 = pltpu.async_copy(pg_t.at[iv0], rv0, s0)
        c1 = pltpu.async_copy(mt_t.at[iv1], rv1, s1)
        c2 = pltpu.async_copy(nm_t.at[iv2], rv2, s2)
        c0.wait()
        c1.wait()
        c2.wait()
        pltpu.sync_copy(rv0, pe_o.at[pl.ds(base, bpw)])
        pltpu.sync_copy(rv1, me_o.at[pl.ds(base, bpw)])
        pltpu.sync_copy(rv2, ne_o.at[pl.ds(base, bpw)])

    return gather_k(emb_page, emb_mat, emb_name, page_idx, mat_idx, name_idx)


def _tc_heads(x, pe, me, ne, pos, dirc, evt, w_small, b_small, w_name, b_name):
    b = x.shape[0]
    bm = 512
    n_small = w_small.shape[1]
    n_name = w_name.shape[1]

    def body(x_r, pe_r, me_r, ne_r, pos_r, dir_r, evt_r,
             ws_r, bs_r, wn_r, bn_r, os_r, on_r):
        hb = jnp.concatenate(
            [x_r[...], pe_r[...], me_r[...], ne_r[...],
             pos_r[...], dir_r[...], evt_r[...]], axis=1)
        os_r[...] = jnp.dot(hb, ws_r[...],
                            preferred_element_type=jnp.float32) + bs_r[...]
        on_r[...] = jnp.dot(hb, wn_r[...],
                            preferred_element_type=jnp.float32) + bn_r[...]

    grid = (b // bm,)
    row = lambda i: (i, 0)
    rep = lambda i: (0, 0)
    out_small, out_name = pl.pallas_call(
        body,
        grid=grid,
        in_specs=[
            pl.BlockSpec((bm, 64), row),
            pl.BlockSpec((bm, 16), row),
            pl.BlockSpec((bm, 16), row),
            pl.BlockSpec((bm, 16), row),
            pl.BlockSpec((bm, 3), row),
            pl.BlockSpec((bm, 1), row),
            pl.BlockSpec((bm, 1), row),
            pl.BlockSpec((117, n_small), rep),
            pl.BlockSpec((1, n_small), rep),
            pl.BlockSpec((117, n_name), rep),
            pl.BlockSpec((1, n_name), rep),
        ],
        out_specs=[
            pl.BlockSpec((bm, n_small), row),
            pl.BlockSpec((bm, n_name), row),
        ],
        out_shape=[
            jax.ShapeDtypeStruct((b, n_small), jnp.float32),
            jax.ShapeDtypeStruct((b, n_name), jnp.float32),
        ],
        compiler_params=pltpu.CompilerParams(
            dimension_semantics=("arbitrary",),
        ),
    )(x, pe, me, ne, pos, dirc, evt, w_small, b_small, w_name, b_name)
    return out_small, out_name


def kernel(x, PageName, MaterialName, Name, Position, Direction, event_type,
           emb_page, emb_mat, emb_name,
           W_time, b_time, W_inputs, b_inputs, W_position, b_position,
           W_velocity, b_velocity, W_event_type, b_event_type,
           W_block_name, b_block_name, W_block_position, b_block_position,
           W_block_direction, b_block_direction):
    pe, me, ne = _sc_gather3(emb_page, emb_mat, emb_name,
                             PageName, MaterialName, Name)

    w_small = jnp.concatenate(
        [W_time, W_inputs, W_position, W_velocity, W_event_type,
         W_block_position, W_block_direction], axis=1)
    b_small = jnp.concatenate(
        [b_time, b_inputs, b_position, b_velocity, b_event_type,
         b_block_position, b_block_direction], axis=0)[None, :]

    out_small, out_name = _tc_heads(
        x, pe, me, ne,
        Position, Direction[:, None], event_type[:, None],
        w_small, b_small, W_block_name, b_block_name[None, :])

    out_time = out_small[:, 0:1]
    out_inputs = out_small[:, 1:4]
    out_position = out_small[:, 4:7]
    out_velocity = out_small[:, 7:10]
    out_event_type = out_small[:, 10:18]
    out_block_position = out_small[:, 18:21]
    out_block_direction = out_small[:, 21:25]
    return (out_time, out_inputs, out_position, out_velocity, out_event_type,
            out_name, out_block_position, out_block_direction)


# speedup vs baseline: 2.2917x; 2.2039x over previous
import functools

import jax
import jax.numpy as jnp
from jax import lax
from jax.experimental import pallas as pl
from jax.experimental.pallas import tpu as pltpu
from jax.experimental.pallas import tpu_sc as plsc


def _sc_gather3(emb_page, emb_mat, emb_name, page_idx, mat_idx, name_idx):
    info = plsc.get_sparse_core_info()
    nw = info.num_cores * info.num_subcores
    b = page_idx.shape[0]
    ed = emb_page.shape[1]
    bpw = b // nw
    mesh = plsc.VectorSubcoreMesh(core_axis_name="c", subcore_axis_name="s")

    @functools.partial(
        pl.kernel,
        mesh=mesh,
        out_type=(
            jax.ShapeDtypeStruct((b, ed), jnp.float32),
            jax.ShapeDtypeStruct((b, ed), jnp.float32),
            jax.ShapeDtypeStruct((b, ed), jnp.float32),
        ),
        scratch_types=[
            pltpu.VMEM((bpw,), jnp.int32),
            pltpu.VMEM((bpw,), jnp.int32),
            pltpu.VMEM((bpw,), jnp.int32),
            pltpu.VMEM((bpw, ed), jnp.float32),
            pltpu.VMEM((bpw, ed), jnp.float32),
            pltpu.VMEM((bpw, ed), jnp.float32),
            pltpu.SemaphoreType.DMA,
            pltpu.SemaphoreType.DMA,
            pltpu.SemaphoreType.DMA,
        ],
        compiler_params=pltpu.CompilerParams(use_tc_tiling_on_sc=False),
    )
    def gather_k(pg_t, mt_t, nm_t, pg_i, mt_i, nm_i, pe_o, me_o, ne_o,
                 iv0, iv1, iv2, rv0, rv1, rv2, s0, s1, s2):
        wid = lax.axis_index("s") * info.num_cores + lax.axis_index("c")
        base = wid * bpw
        pltpu.sync_copy(pg_i.at[pl.ds(base, bpw)], iv0)
        pltpu.sync_copy(mt_i.at[pl.ds(base, bpw)], iv1)
        pltpu.sync_copy(nm_i.at[pl.ds(base, bpw)], iv2)
        c0 = pltpu.async_copy(pg_t.at[iv0], rv0, s0)
        c1 = pltpu.async_copy(mt_t.at[iv1], rv1, s1)
        c2 = pltpu.async_copy(nm_t.at[iv2], rv2, s2)
        c0.wait()
        c1.wait()
        c2.wait()
        pltpu.sync_copy(rv0, pe_o.at[pl.ds(base, bpw)])
        pltpu.sync_copy(rv1, me_o.at[pl.ds(base, bpw)])
        pltpu.sync_copy(rv2, ne_o.at[pl.ds(base, bpw)])

    return gather_k(emb_page, emb_mat, emb_name, page_idx, mat_idx, name_idx)


_HEAD_DIMS = (1, 3, 3, 3, 8, 1000, 3, 4)


def _tc_heads(xT, pe, me, ne, posT, dirT, evtT, wT, bT):
    b = xT.shape[1]
    bm = 512
    grid = (b // bm,)

    def body(xT_r, pe_r, me_r, ne_r, posT_r, dirT_r, evtT_r, wT_r, bT_r,
             *out_refs):
        hT = jnp.concatenate(
            [xT_r[...], pe_r[...].T, me_r[...].T, ne_r[...].T,
             posT_r[...], dirT_r[...], evtT_r[...]], axis=0)
        acc = jax.lax.dot_general(
            wT_r[...], hT, (((1,), (0,)), ((), ())),
            preferred_element_type=jnp.float32) + bT_r[...]
        off = 0
        for ref, dim in zip(out_refs, _HEAD_DIMS):
            ref[...] = acc[off:off + dim, :]
            off += dim

    col = lambda i: (0, i)
    row = lambda i: (i, 0)
    rep = lambda i: (0, 0)
    outs = pl.pallas_call(
        body,
        grid=grid,
        in_specs=[
            pl.BlockSpec((64, bm), col),
            pl.BlockSpec((bm, 16), row),
            pl.BlockSpec((bm, 16), row),
            pl.BlockSpec((bm, 16), row),
            pl.BlockSpec((3, bm), col),
            pl.BlockSpec((1, bm), col),
            pl.BlockSpec((1, bm), col),
            pl.BlockSpec((1025, 117), rep),
            pl.BlockSpec((1025, 1), rep),
        ],
        out_specs=[pl.BlockSpec((d, bm), col) for d in _HEAD_DIMS],
        out_shape=[jax.ShapeDtypeStruct((d, b), jnp.float32)
                   for d in _HEAD_DIMS],
        compiler_params=pltpu.CompilerParams(
            dimension_semantics=("arbitrary",),
        ),
    )(xT, pe, me, ne, posT, dirT, evtT, wT, bT)
    return outs


def kernel(x, PageName, MaterialName, Name, Position, Direction, event_type,
           emb_page, emb_mat, emb_name,
           W_time, b_time, W_inputs, b_inputs, W_position, b_position,
           W_velocity, b_velocity, W_event_type, b_event_type,
           W_block_name, b_block_name, W_block_position, b_block_position,
           W_block_direction, b_block_direction):
    pe, me, ne = _sc_gather3(emb_page, emb_mat, emb_name,
                             PageName, MaterialName, Name)

    wT = jnp.concatenate(
        [W_time.T, W_inputs.T, W_position.T, W_velocity.T, W_event_type.T,
         W_block_name.T, W_block_position.T, W_block_direction.T], axis=0)
    bT = jnp.concatenate(
        [b_time, b_inputs, b_position, b_velocity, b_event_type,
         b_block_name, b_block_position, b_block_direction], axis=0)[:, None]

    outs = _tc_heads(
        x.T, pe, me, ne,
        Position.T, Direction[None, :], event_type[None, :], wT, bT)
    return tuple(o.T for o in outs)


# speedup vs baseline: 3.8644x; 1.6863x over previous
import functools

import jax
import jax.numpy as jnp
from jax import lax
from jax.experimental import pallas as pl
from jax.experimental.pallas import tpu as pltpu
from jax.experimental.pallas import tpu_sc as plsc


def _sc_gather3(pgT, mtT, nmT, page_idx, mat_idx, name_idx):
    info = plsc.get_sparse_core_info()
    b = page_idx.shape[0]
    ed = pgT.shape[0]
    v_pg, v_mt, v_nm = pgT.shape[1], mtT.shape[1], nmT.shape[1]
    half = b // info.num_cores
    mesh = plsc.VectorSubcoreMesh(core_axis_name="c", subcore_axis_name="s")

    @functools.partial(
        pl.kernel,
        mesh=mesh,
        out_type=(
            jax.ShapeDtypeStruct((ed, b), jnp.float32),
            jax.ShapeDtypeStruct((ed, b), jnp.float32),
            jax.ShapeDtypeStruct((ed, b), jnp.float32),
        ),
        scratch_types=[
            pltpu.VMEM((v_nm,), jnp.float32),
            pltpu.VMEM((v_pg,), jnp.float32),
            pltpu.VMEM((v_mt,), jnp.float32),
            pltpu.VMEM((half,), jnp.int32),
            pltpu.VMEM((half,), jnp.float32),
            pltpu.SemaphoreType.DMA,
        ],
        compiler_params=pltpu.CompilerParams(use_tc_tiling_on_sc=True,
                                             needs_layout_passes=False),
    )
    def gather_k(pgT_h, mtT_h, nmT_h, pg_i, mt_i, nm_i, pe_o, me_o, ne_o,
                 row_nm, row_pg, row_mt, idxb, outb, sem):
        c = lax.axis_index("c")
        s = lax.axis_index("s")
        col0 = c * half
        nm_dma = pltpu.async_copy(nmT_h.at[s], row_nm, sem)

        def gather_rows(row_ref, idx_ref, out_ref):
            pltpu.sync_copy(idx_ref.at[pl.ds(col0, half)], idxb)

            def step(j, carry):
                iv = idxb[pl.ds(j * 16, 16)]
                outb[pl.ds(j * 16, 16)] = plsc.load_gather(row_ref, [iv])
                return carry

            lax.fori_loop(0, half // 16, step, 0, unroll=8)
            pltpu.sync_copy(outb, out_ref.at[s, pl.ds(col0, half)])

        pltpu.sync_copy(pgT_h.at[s], row_pg)
        gather_rows(row_pg, pg_i, pe_o)
        pltpu.sync_copy(mtT_h.at[s], row_mt)
        gather_rows(row_mt, mt_i, me_o)
        nm_dma.wait()
        gather_rows(row_nm, nm_i, ne_o)

    return gather_k(pgT, mtT, nmT, page_idx, mat_idx, name_idx)


_HEAD_DIMS = (1, 3, 3, 3, 8, 1000, 3, 4)


def _tc_heads(xT, peT, meT, neT, posT, dirT, evtT, wsT, bsT, wnT, bnT):
    b = xT.shape[1]
    bm = 512
    grid = (b // bm,)

    def body(xT_r, peT_r, meT_r, neT_r, posT_r, dirT_r, evtT_r,
             wsT_r, bsT_r, wnT_r, bnT_r, *out_refs):
        hT = jnp.concatenate(
            [xT_r[...], peT_r[...], meT_r[...], neT_r[...],
             posT_r[...], dirT_r[...], evtT_r[...]], axis=0)
        acc_s = jax.lax.dot_general(
            wsT_r[...], hT, (((1,), (0,)), ((), ())),
            preferred_element_type=jnp.float32) + bsT_r[...]
        acc_n = jax.lax.dot_general(
            wnT_r[...], hT, (((1,), (0,)), ((), ())),
            preferred_element_type=jnp.float32) + bnT_r[...]
        off = 0
        for ref, dim in zip(out_refs, _HEAD_DIMS):
            if dim == 1000:
                ref[...] = acc_n
            else:
                ref[...] = acc_s[off:off + dim, :]
                off += dim

    col = lambda i: (0, i)
    rep = lambda i: (0, 0)
    outs = pl.pallas_call(
        body,
        grid=grid,
        in_specs=[
            pl.BlockSpec((64, bm), col),
            pl.BlockSpec((16, bm), col),
            pl.BlockSpec((16, bm), col),
            pl.BlockSpec((16, bm), col),
            pl.BlockSpec((3, bm), col),
            pl.BlockSpec((1, bm), col),
            pl.BlockSpec((1, bm), col),
            pl.BlockSpec((25, 117), rep),
            pl.BlockSpec((25, 1), rep),
            pl.BlockSpec((1000, 117), rep),
            pl.BlockSpec((1000, 1), rep),
        ],
        out_specs=[pl.BlockSpec((d, bm), col) for d in _HEAD_DIMS],
        out_shape=[jax.ShapeDtypeStruct((d, b), jnp.float32)
                   for d in _HEAD_DIMS],
        compiler_params=pltpu.CompilerParams(
            dimension_semantics=("arbitrary",),
        ),
    )(xT, peT, meT, neT, posT, dirT, evtT, wsT, bsT, wnT, bnT)
    return outs


def kernel(x, PageName, MaterialName, Name, Position, Direction, event_type,
           emb_page, emb_mat, emb_name,
           W_time, b_time, W_inputs, b_inputs, W_position, b_position,
           W_velocity, b_velocity, W_event_type, b_event_type,
           W_block_name, b_block_name, W_block_position, b_block_position,
           W_block_direction, b_block_direction):
    peT, meT, neT = _sc_gather3(emb_page.T, emb_mat.T, emb_name.T,
                                PageName, MaterialName, Name)

    wsT = jnp.concatenate(
        [W_time.T, W_inputs.T, W_position.T, W_velocity.T, W_event_type.T,
         W_block_position.T, W_block_direction.T], axis=0)
    bsT = jnp.concatenate(
        [b_time, b_inputs, b_position, b_velocity, b_event_type,
         b_block_position, b_block_direction], axis=0)[:, None]

    outs = _tc_heads(
        x.T, peT, meT, neT,
        Position.T, Direction[None, :], event_type[None, :],
        wsT, bsT, W_block_name.T, b_block_name[:, None])
    o_time, o_inputs, o_position, o_velocity, o_event, o_name, o_bpos, o_bdir = outs
    return (o_time.T, o_inputs.T, o_position.T, o_velocity.T, o_event.T,
            o_name.T, o_bpos.T, o_bdir.T)


# speedup vs baseline: 5.2980x; 1.3710x over previous
import functools

import jax
import jax.numpy as jnp
from jax import lax
from jax.experimental import pallas as pl
from jax.experimental.pallas import tpu as pltpu
from jax.experimental.pallas import tpu_sc as plsc


def _sc_gather3(pgT, mtT, nmT, page_idx, mat_idx, name_idx):
    info = plsc.get_sparse_core_info()
    b = page_idx.shape[0]
    ed = pgT.shape[0]
    v_pg, v_mt, v_nm = pgT.shape[1], mtT.shape[1], nmT.shape[1]
    half = b // info.num_cores
    mesh = plsc.VectorSubcoreMesh(core_axis_name="c", subcore_axis_name="s")

    @functools.partial(
        pl.kernel,
        mesh=mesh,
        out_type=(
            jax.ShapeDtypeStruct((ed, b), jnp.float32),
            jax.ShapeDtypeStruct((ed, b), jnp.float32),
            jax.ShapeDtypeStruct((ed, b), jnp.float32),
        ),
        scratch_types=[
            pltpu.VMEM((v_nm,), jnp.float32),
            pltpu.VMEM((v_pg,), jnp.float32),
            pltpu.VMEM((v_mt,), jnp.float32),
            pltpu.VMEM((half,), jnp.int32),
            pltpu.VMEM((half,), jnp.float32),
            pltpu.VMEM((half,), jnp.float32),
            pltpu.SemaphoreType.DMA,
            pltpu.SemaphoreType.DMA,
            pltpu.SemaphoreType.DMA,
        ],
        compiler_params=pltpu.CompilerParams(use_tc_tiling_on_sc=True,
                                             needs_layout_passes=False),
    )
    def gather_k(pgT_h, mtT_h, nmT_h, pg_i, mt_i, nm_i, pe_o, me_o, ne_o,
                 row_nm, row_pg, row_mt, idxb, outb, outb2,
                 sem, sem2, sem3):
        c = lax.axis_index("c")
        s = lax.axis_index("s")
        col0 = c * half
        nm_dma = pltpu.async_copy(nmT_h.at[s], row_nm, sem)

        def gather_rows(row_ref, ob):
            @plsc.parallel_loop(0, half // 16, unroll=4)
            def step(j):
                iv = idxb[pl.ds(j * 16, 16)]
                ob[pl.ds(j * 16, 16)] = plsc.load_gather(row_ref, [iv])

        pltpu.sync_copy(pgT_h.at[s], row_pg)
        pltpu.sync_copy(pg_i.at[pl.ds(col0, half)], idxb)
        gather_rows(row_pg, outb)
        pe_dma = pltpu.async_copy(outb, pe_o.at[s, pl.ds(col0, half)], sem2)

        pltpu.sync_copy(mtT_h.at[s], row_mt)
        pltpu.sync_copy(mt_i.at[pl.ds(col0, half)], idxb)
        gather_rows(row_mt, outb2)
        me_dma = pltpu.async_copy(outb2, me_o.at[s, pl.ds(col0, half)], sem3)

        pltpu.sync_copy(nm_i.at[pl.ds(col0, half)], idxb)
        nm_dma.wait()
        pe_dma.wait()
        gather_rows(row_nm, outb)
        pltpu.sync_copy(outb, ne_o.at[s, pl.ds(col0, half)])
        me_dma.wait()

    return gather_k(pgT, mtT, nmT, page_idx, mat_idx, name_idx)


_HEAD_DIMS = (1, 3, 3, 3, 8, 1000, 3, 4)


def _tc_heads(xT, peT, meT, neT, posT, dirT, evtT, ws_list, bsT, wnT, bnT):
    b = xT.shape[1]
    bm = 2048
    grid = (b // bm,)

    def body(xT_r, peT_r, meT_r, neT_r, posT_r, dirT_r, evtT_r,
             w0_r, w1_r, w2_r, w3_r, w4_r, w5_r, w6_r,
             bsT_r, wnT_r, bnT_r, *out_refs):
        hT = jnp.concatenate(
            [xT_r[...], peT_r[...], meT_r[...], neT_r[...],
             posT_r[...], dirT_r[...], evtT_r[...]], axis=0)
        wsT = jnp.concatenate(
            [w0_r[...], w1_r[...], w2_r[...], w3_r[...],
             w4_r[...], w5_r[...], w6_r[...]], axis=0)
        acc_s = jax.lax.dot_general(
            wsT, hT, (((1,), (0,)), ((), ())),
            preferred_element_type=jnp.float32) + bsT_r[...]
        acc_n = jax.lax.dot_general(
            wnT_r[...], hT, (((1,), (0,)), ((), ())),
            preferred_element_type=jnp.float32) + bnT_r[...]
        off = 0
        for ref, dim in zip(out_refs, _HEAD_DIMS):
            if dim == 1000:
                ref[...] = acc_n
            else:
                ref[...] = acc_s[off:off + dim, :]
                off += dim

    col = lambda i: (0, i)
    rep = lambda i: (0, 0)
    outs = pl.pallas_call(
        body,
        grid=grid,
        in_specs=[
            pl.BlockSpec((64, bm), col),
            pl.BlockSpec((16, bm), col),
            pl.BlockSpec((16, bm), col),
            pl.BlockSpec((16, bm), col),
            pl.BlockSpec((3, bm), col),
            pl.BlockSpec((1, bm), col),
            pl.BlockSpec((1, bm), col),
            pl.BlockSpec((1, 117), rep),
            pl.BlockSpec((3, 117), rep),
            pl.BlockSpec((3, 117), rep),
            pl.BlockSpec((3, 117), rep),
            pl.BlockSpec((8, 117), rep),
            pl.BlockSpec((3, 117), rep),
            pl.BlockSpec((4, 117), rep),
            pl.BlockSpec((25, 1), rep),
            pl.BlockSpec((1000, 117), rep),
            pl.BlockSpec((1000, 1), rep),
        ],
        out_specs=[pl.BlockSpec((d, bm), col) for d in _HEAD_DIMS],
        out_shape=[jax.ShapeDtypeStruct((d, b), jnp.float32)
                   for d in _HEAD_DIMS],
        compiler_params=pltpu.CompilerParams(
            dimension_semantics=("arbitrary",),
        ),
    )(xT, peT, meT, neT, posT, dirT, evtT, *ws_list, bsT, wnT, bnT)
    return outs


def kernel(x, PageName, MaterialName, Name, Position, Direction, event_type,
           emb_page, emb_mat, emb_name,
           W_time, b_time, W_inputs, b_inputs, W_position, b_position,
           W_velocity, b_velocity, W_event_type, b_event_type,
           W_block_name, b_block_name, W_block_position, b_block_position,
           W_block_direction, b_block_direction):
    peT, meT, neT = _sc_gather3(emb_page.T, emb_mat.T, emb_name.T,
                                PageName, MaterialName, Name)

    ws_list = [W_time.T, W_inputs.T, W_position.T, W_velocity.T,
               W_event_type.T, W_block_position.T, W_block_direction.T]
    bsT = jnp.concatenate(
        [b_time, b_inputs, b_position, b_velocity, b_event_type,
         b_block_position, b_block_direction], axis=0)[:, None]

    outs = _tc_heads(
        x.T, peT, meT, neT,
        Position.T, Direction[None, :], event_type[None, :],
        ws_list, bsT, W_block_name.T, b_block_name[:, None])
    o_time, o_inputs, o_position, o_velocity, o_event, o_name, o_bpos, o_bdir = outs
    return (o_time.T, o_inputs.T, o_position.T, o_velocity.T, o_event.T,
            o_name.T, o_bpos.T, o_bdir.T)
